# baseline (speedup 1.0000x reference)
import functools

import jax
import jax.numpy as jnp
from jax import lax
from jax.experimental import pallas as pl
from jax.experimental.pallas import tpu as pltpu
from jax.experimental.pallas import tpu_sc as plsc

_B, _S, _D = 4, 8192, 1024
_ROWS_OUT = _B * (_S // 2)
_NW = 32
_ROWS_PER_W = _ROWS_OUT // _NW
_CHUNK_ROWS = 8
_CHUNKS = _ROWS_PER_W // _CHUNK_ROWS
_NBUF = 4
_LANES = 16


def _avg_pool_sc(x2):
    mesh = plsc.VectorSubcoreMesh(core_axis_name="c", subcore_axis_name="s")

    @functools.partial(
        pl.kernel,
        mesh=mesh,
        out_type=jax.ShapeDtypeStruct((_ROWS_OUT, _D), jnp.float32),
        scratch_types=(
            [pltpu.VMEM((2 * _CHUNK_ROWS, _D), jnp.float32)] * _NBUF
            + [pltpu.VMEM((_CHUNK_ROWS, _D), jnp.float32)] * _NBUF
            + [pltpu.SemaphoreType.DMA] * (2 * _NBUF)
        ),
    )
    def k(x_hbm, o_hbm, *bufs):
        in_bufs = bufs[:_NBUF]
        out_bufs = bufs[_NBUF:2 * _NBUF]
        sin = bufs[2 * _NBUF:3 * _NBUF]
        sout = bufs[3 * _NBUF:4 * _NBUF]

        wid = lax.axis_index("s") * 2 + lax.axis_index("c")
        base_in = wid * (_ROWS_PER_W * 2)
        base_out = wid * _ROWS_PER_W

        def in_copy(g, b):
            return pltpu.make_async_copy(
                x_hbm.at[pl.ds(base_in + g * 2 * _CHUNK_ROWS, 2 * _CHUNK_ROWS)],
                in_bufs[b], sin[b])

        def out_copy(g, b):
            return pltpu.make_async_copy(
                out_bufs[b],
                o_hbm.at[pl.ds(base_out + g * _CHUNK_ROWS, _CHUNK_ROWS)],
                sout[b])

        for b in range(_NBUF):
            in_copy(b, b).start()

        def outer(gg, carry):
            for b in range(_NBUF):
                g = gg * _NBUF + b

                in_copy(g, b).wait()

                @pl.when(g >= _NBUF)
                def _drain_prev():
                    out_copy(g - _NBUF, b).wait()

                out_v = out_bufs[b]
                in_v = in_bufs[b]

                @plsc.parallel_loop(0, _CHUNK_ROWS * (_D // _LANES), unroll=8)
                def vec_body(j):
                    row = j >> 6
                    col = (j & (_D // _LANES - 1)) * _LANES
                    a = in_v[2 * row, pl.ds(col, _LANES)]
                    bb = in_v[2 * row + 1, pl.ds(col, _LANES)]
                    out_v[row, pl.ds(col, _LANES)] = (a + bb) * 0.5

                out_copy(g, b).start()

                @pl.when(g + _NBUF < _CHUNKS)
                def _start_next():
                    in_copy(g + _NBUF, b).start()
            return carry

        lax.fori_loop(0, _CHUNKS // _NBUF, outer, 0)
        for b in range(_NBUF):
            out_copy(_CHUNKS - _NBUF + b, b).wait()

    return k(x2)


def kernel(x):
    x2 = x.reshape(_ROWS_OUT * 2, _D)
    of = _avg_pool_sc(x2)
    return of.reshape(_B, _S // 2, _D)
